# Initial kernel scaffold; baseline (speedup 1.0000x reference)
import jax
import jax.numpy as jnp
from jax.experimental import pallas as pl


def kernel(x, edge_index, W1, b1, g1, be1, W2, b2, g2, be2, Wc, bc, gc, bec, Wl, bl, Wf, bf):
    raise NotImplementedError("write your pallas kernel here")


# speedup vs baseline: 6.0052x; 6.0052x over previous
import functools

import jax
import jax.numpy as jnp
from jax import lax
from jax.experimental import pallas as pl
from jax.experimental.pallas import tpu as pltpu
from jax.experimental.pallas import tpu_sc as plsc

N = 10000
N_OCC = 3
NK = 19
NP = 6
NF = 44
FP = 48
SF = 25
SFP = 32
Q = N * NP
EPS = 1e-5
SHIFT = 0.6931

NWORK = 32
PW = 1880
QPAD = NWORK * PW
B = 40
NITER = PW // B
BR = B * NK

TN = 400
NTILES = N // TN


def _sc_body(table_hbm, idx_hbm, out_hbm, idx_v, rows_v, x_v, sem):
    wid = lax.axis_index("s") * 2 + lax.axis_index("c")
    base = wid * PW

    @pl.loop(0, NITER)
    def _(it):
        q0 = base + it * B
        pltpu.sync_copy(idx_hbm.at[pl.ds(q0 * NK, BR)], idx_v)
        pltpu.async_copy(table_hbm.at[idx_v], rows_v, sem).wait()

        @pl.loop(0, B)
        def _(j):
            r0 = j * NK
            for c in range(FP // 16):
                sl = pl.ds(c * 16, 16)
                acc = rows_v[r0, sl]
                for r in range(1, NK):
                    acc = acc + rows_v[r0 + r, sl]
                x_v[j, sl] = acc

        pltpu.sync_copy(x_v, out_hbm.at[pl.ds(q0, B)])


def _gather_sum(table, idx_flat):
    mesh = plsc.VectorSubcoreMesh(core_axis_name="c", subcore_axis_name="s")
    kfn = pl.kernel(
        _sc_body,
        out_type=jax.ShapeDtypeStruct((QPAD, FP), jnp.float32),
        mesh=mesh,
        compiler_params=pltpu.CompilerParams(use_tc_tiling_on_sc=False),
        scratch_types=[
            pltpu.VMEM((BR,), jnp.int32),
            pltpu.VMEM((BR, FP), jnp.float32),
            pltpu.VMEM((B, FP), jnp.float32),
            pltpu.SemaphoreType.DMA,
        ],
    )
    return kfn(table, idx_flat)


def _mm_a_body(x_ref, w_ref, o_ref):
    o_ref[...] = jnp.dot(x_ref[...], w_ref[...],
                         preferred_element_type=jnp.float32)


def _bn_sum(xrefs, b_ref, g_ref, be_ref):
    b = b_ref[0:1, :]
    xs = [r[...] + b for r in xrefs]
    m = xs[0]
    for xi in xs[1:]:
        m = m + xi
    m = m * (1.0 / NP)
    var = (xs[0] - m) ** 2
    for xi in xs[1:]:
        var = var + (xi - m) ** 2
    var = var * (1.0 / NP)
    inv = lax.rsqrt(var + EPS)
    g = g_ref[0:1, :]
    be = be_ref[0:1, :]
    h = (xs[0] - m) * inv * g + be
    for xi in xs[1:]:
        h = h + (xi - m) * inv * g + be
    return h


def _bn_mm_body(x0, x1, x2, x3, x4, x5, b_ref, g_ref, be_ref, w_ref, o_ref):
    h = _bn_sum((x0, x1, x2, x3, x4, x5), b_ref, g_ref, be_ref)
    o_ref[...] = jnp.dot(h, w_ref[...], preferred_element_type=jnp.float32)


def _head_body(x0, x1, x2, x3, x4, x5, b_ref, g_ref, be_ref,
               wc_ref, bc_ref, gc_ref, bec_ref, wl_ref, bl_ref, wfb_ref,
               o_ref, acc_ref):
    i = pl.program_id(0)

    @pl.when(i == 0)
    def _():
        acc_ref[...] = jnp.zeros_like(acc_ref)

    h = _bn_sum((x0, x1, x2, x3, x4, x5), b_ref, g_ref, be_ref)
    hc = jnp.dot(h, wc_ref[...], preferred_element_type=jnp.float32)
    hc = hc + bc_ref[0:1, :]
    lane = lax.broadcasted_iota(jnp.int32, hc.shape, 1)
    mask = lane < SF
    mu = jnp.sum(hc, axis=-1, keepdims=True) * (1.0 / SF)
    d = jnp.where(mask, hc - mu, 0.0)
    sig = jnp.sum(d * d, axis=-1, keepdims=True) * (1.0 / SF)
    hn = d * lax.rsqrt(sig + EPS) * gc_ref[0:1, :] + bec_ref[0:1, :]
    sp = jnp.maximum(hn, 0.0) + jnp.log(1.0 + jnp.exp(-jnp.abs(hn))) - SHIFT
    sp = jnp.where(mask, sp, 0.0)
    hl = jnp.dot(sp, wl_ref[...], preferred_element_type=jnp.float32)
    hl = hl + bl_ref[0:1, :]
    acc_ref[0:1, 0:SFP] += jnp.sum(hl, axis=0, keepdims=True)

    @pl.when(i == NTILES - 1)
    def _():
        gmean = acc_ref[0:1, 0:SFP] * (1.0 / N)
        val = jnp.sum(gmean * wfb_ref[0:1, :]) + wfb_ref[1, 0]
        o_ref[...] = jnp.full((8, 128), val, jnp.float32)


def _x_specs():
    return [pl.BlockSpec((TN, FP), functools.partial(
        lambda p, i: (p * NTILES + i, 0), p)) for p in range(NP)]


def _vec_spec():
    return pl.BlockSpec((8, FP), lambda i: (0, 0))


def _pad_row(v, width):
    out = jnp.zeros((8, width), jnp.float32)
    return out.at[0, : v.shape[0]].set(v)


def kernel(x, edge_index, W1, b1, g1, be1, W2, b2, g2, be2,
           Wc, bc, gc, bec, Wl, bl, Wf, bf):
    src = edge_index[0].astype(jnp.int32)
    src3 = src.reshape(N, NP, NK)
    idx3 = src3 * NK + jnp.arange(NK, dtype=jnp.int32)[None, None, :]
    idxp = jnp.transpose(idx3, (1, 0, 2)).reshape(Q, NK)
    idxp = jnp.concatenate(
        [idxp, jnp.zeros((QPAD - Q, NK), jnp.int32)], axis=0)
    idx_flat = idxp.reshape(-1)

    def wconv(W, F):
        Wr = W.reshape(NK, F, NF).transpose(1, 0, 2)
        Wr = jnp.pad(Wr, ((0, 0), (0, 0), (0, FP - NF)))
        return Wr.reshape(F, NK * FP)

    W1r = jnp.pad(wconv(W1, N_OCC), ((0, 8 - N_OCC), (0, 0)))
    xp = jnp.pad(x, ((0, 0), (0, 8 - N_OCC)))
    W2r = jnp.pad(wconv(W2, NF), ((0, FP - NF), (0, 0)))

    b1p, g1p, be1p = _pad_row(b1, FP), _pad_row(g1, FP), _pad_row(be1, FP)
    b2p, g2p, be2p = _pad_row(b2, FP), _pad_row(g2, FP), _pad_row(be2, FP)
    Wcp = jnp.pad(Wc, ((0, FP - NF), (0, SFP - SF)))
    bcp, gcp, becp = _pad_row(bc, SFP), _pad_row(gc, SFP), _pad_row(bec, SFP)
    Wlp = jnp.pad(Wl, ((0, SFP - SF), (0, SFP - SF)))
    blp = _pad_row(bl, SFP)
    wfb = jnp.zeros((8, SFP), jnp.float32)
    wfb = wfb.at[0, :SF].set(Wf[:, 0]).at[1, 0].set(bf[0])

    G1 = pl.pallas_call(
        _mm_a_body,
        grid=(NTILES,),
        in_specs=[pl.BlockSpec((TN, 8), lambda i: (i, 0)),
                  pl.BlockSpec((8, NK * FP), lambda i: (0, 0))],
        out_specs=pl.BlockSpec((TN, NK * FP), lambda i: (i, 0)),
        out_shape=jax.ShapeDtypeStruct((N, NK * FP), jnp.float32),
    )(xp, W1r)

    X1 = _gather_sum(G1.reshape(N * NK, FP), idx_flat)

    G2 = pl.pallas_call(
        _bn_mm_body,
        grid=(NTILES,),
        in_specs=_x_specs() + [_vec_spec()] * 3
        + [pl.BlockSpec((FP, NK * FP), lambda i: (0, 0))],
        out_specs=pl.BlockSpec((TN, NK * FP), lambda i: (i, 0)),
        out_shape=jax.ShapeDtypeStruct((N, NK * FP), jnp.float32),
    )(X1, X1, X1, X1, X1, X1, b1p, g1p, be1p, W2r)

    X2 = _gather_sum(G2.reshape(N * NK, FP), idx_flat)

    sfv = pl.BlockSpec((8, SFP), lambda i: (0, 0))
    out = pl.pallas_call(
        _head_body,
        grid=(NTILES,),
        in_specs=_x_specs() + [_vec_spec()] * 3
        + [pl.BlockSpec((FP, SFP), lambda i: (0, 0)), sfv, sfv, sfv,
           pl.BlockSpec((SFP, SFP), lambda i: (0, 0)), sfv, sfv],
        out_specs=pl.BlockSpec((8, 128), lambda i: (0, 0)),
        out_shape=jax.ShapeDtypeStruct((8, 128), jnp.float32),
        scratch_shapes=[pltpu.VMEM((8, 128), jnp.float32)],
    )(X2, X2, X2, X2, X2, X2, b2p, g2p, be2p,
      Wcp, bcp, gcp, becp, Wlp, blp, wfb)

    return out[0:1, 0]


# speedup vs baseline: 7.6404x; 1.2723x over previous
import functools

import jax
import jax.numpy as jnp
from jax import lax
from jax.experimental import pallas as pl
from jax.experimental.pallas import tpu as pltpu
from jax.experimental.pallas import tpu_sc as plsc

N = 10000
N_OCC = 3
NK = 19
NP = 6
NF = 44
FP = 64
SF = 25
SFP = 32
Q = N * NP
EPS = 1e-5
SHIFT = 0.6931

NWORK = 32
PW = 1880
QPAD = NWORK * PW
B = 40
NITER = PW // B
BR = B * NK

TN = 400
NTILES = N // TN


def _sc_body(table_hbm, idx_hbm, out_hbm, idx_v, rows_v, x_v, g0, g1, o0, o1):
    wid = lax.axis_index("s") * 2 + lax.axis_index("c")
    base = wid * PW
    pltpu.sync_copy(idx_hbm.at[pl.ds(base * NK, PW * NK)], idx_v)

    def gather_start(it, buf, sem):
        pltpu.make_async_copy(
            table_hbm.at[idx_v.at[pl.ds(it * BR, BR)]],
            rows_v.at[buf], sem).start()

    def gather_wait(buf, sem):
        pltpu.make_async_copy(
            table_hbm.at[idx_v.at[pl.ds(0, BR)]],
            rows_v.at[buf], sem).wait()

    def out_start(it, buf, sem):
        pltpu.make_async_copy(
            x_v.at[buf], out_hbm.at[pl.ds(base + it * B, B)], sem).start()

    def out_wait(buf, sem):
        pltpu.make_async_copy(
            x_v.at[buf], out_hbm.at[pl.ds(base, B)], sem).wait()

    def accumulate(buf):
        @pl.loop(0, B)
        def _(j):
            r0 = j * NK
            for c in range(FP // 32):
                sl = pl.ds(c * 32, 32)
                acc = rows_v[buf, r0, sl]
                for r in range(1, NK):
                    acc = acc + rows_v[buf, r0 + r, sl]
                x_v[buf, j, sl] = acc

    gather_start(0, 0, g0)

    @pl.loop(0, (NITER - 1) // 2)
    def _(t):
        it0 = 2 * t
        gather_start(it0 + 1, 1, g1)
        gather_wait(0, g0)

        @pl.when(t > 0)
        def _():
            out_wait(0, o0)

        accumulate(0)
        out_start(it0, 0, o0)

        it1 = it0 + 1
        gather_start(it1 + 1, 0, g0)
        gather_wait(1, g1)

        @pl.when(t > 0)
        def _():
            out_wait(1, o1)

        accumulate(1)
        out_start(it1, 1, o1)

    gather_wait(0, g0)
    out_wait(0, o0)
    accumulate(0)
    out_start(NITER - 1, 0, o0)
    out_wait(0, o0)
    out_wait(1, o1)


def _gather_sum(table, idx_flat):
    mesh = plsc.VectorSubcoreMesh(core_axis_name="c", subcore_axis_name="s")
    kfn = pl.kernel(
        _sc_body,
        out_type=jax.ShapeDtypeStruct((QPAD, FP), jnp.bfloat16),
        mesh=mesh,
        compiler_params=pltpu.CompilerParams(use_tc_tiling_on_sc=False),
        scratch_types=[
            pltpu.VMEM((PW * NK,), jnp.int32),
            pltpu.VMEM((2, BR, FP), jnp.bfloat16),
            pltpu.VMEM((2, B, FP), jnp.bfloat16),
            pltpu.SemaphoreType.DMA,
            pltpu.SemaphoreType.DMA,
            pltpu.SemaphoreType.DMA,
            pltpu.SemaphoreType.DMA,
        ],
    )
    return kfn(table, idx_flat)


def _mm_a_body(x_ref, w_ref, o_ref):
    o_ref[...] = jnp.dot(x_ref[...], w_ref[...],
                         preferred_element_type=jnp.float32
                         ).astype(jnp.bfloat16)


def _bn_sum(xrefs, b_ref, g_ref, be_ref):
    b = b_ref[0:1, :]
    xs = [r[...].astype(jnp.float32) + b for r in xrefs]
    m = xs[0]
    for xi in xs[1:]:
        m = m + xi
    m = m * (1.0 / NP)
    var = (xs[0] - m) ** 2
    for xi in xs[1:]:
        var = var + (xi - m) ** 2
    var = var * (1.0 / NP)
    inv = lax.rsqrt(var + EPS)
    g = g_ref[0:1, :]
    be = be_ref[0:1, :]
    h = (xs[0] - m) * inv * g + be
    for xi in xs[1:]:
        h = h + (xi - m) * inv * g + be
    return h


def _bn_mm_body(x0, x1, x2, x3, x4, x5, b_ref, g_ref, be_ref, w_ref, o_ref):
    h = _bn_sum((x0, x1, x2, x3, x4, x5), b_ref, g_ref, be_ref)
    o_ref[...] = jnp.dot(h, w_ref[...], preferred_element_type=jnp.float32
                         ).astype(jnp.bfloat16)


def _head_body(x0, x1, x2, x3, x4, x5, b_ref, g_ref, be_ref,
               wc_ref, bc_ref, gc_ref, bec_ref, wl_ref, bl_ref, wfb_ref,
               o_ref, acc_ref):
    i = pl.program_id(0)

    @pl.when(i == 0)
    def _():
        acc_ref[...] = jnp.zeros_like(acc_ref)

    h = _bn_sum((x0, x1, x2, x3, x4, x5), b_ref, g_ref, be_ref)
    hc = jnp.dot(h, wc_ref[...], preferred_element_type=jnp.float32)
    hc = hc + bc_ref[0:1, :]
    lane = lax.broadcasted_iota(jnp.int32, hc.shape, 1)
    mask = lane < SF
    mu = jnp.sum(hc, axis=-1, keepdims=True) * (1.0 / SF)
    d = jnp.where(mask, hc - mu, 0.0)
    sig = jnp.sum(d * d, axis=-1, keepdims=True) * (1.0 / SF)
    hn = d * lax.rsqrt(sig + EPS) * gc_ref[0:1, :] + bec_ref[0:1, :]
    sp = jnp.maximum(hn, 0.0) + jnp.log(1.0 + jnp.exp(-jnp.abs(hn))) - SHIFT
    sp = jnp.where(mask, sp, 0.0)
    hl = jnp.dot(sp, wl_ref[...], preferred_element_type=jnp.float32)
    hl = hl + bl_ref[0:1, :]
    acc_ref[0:1, 0:SFP] += jnp.sum(hl, axis=0, keepdims=True)

    @pl.when(i == NTILES - 1)
    def _():
        gmean = acc_ref[0:1, 0:SFP] * (1.0 / N)
        val = jnp.sum(gmean * wfb_ref[0:1, :]) + wfb_ref[1, 0]
        o_ref[...] = jnp.full((8, 128), val, jnp.float32)


def _x_specs():
    return [pl.BlockSpec((TN, FP), functools.partial(
        lambda p, i: (p * NTILES + i, 0), p)) for p in range(NP)]


def _vec_spec():
    return pl.BlockSpec((8, FP), lambda i: (0, 0))


def _pad_row(v, width):
    out = jnp.zeros((8, width), jnp.float32)
    return out.at[0, : v.shape[0]].set(v)


def kernel(x, edge_index, W1, b1, g1, be1, W2, b2, g2, be2,
           Wc, bc, gc, bec, Wl, bl, Wf, bf):
    src = edge_index[0].astype(jnp.int32)
    src3 = src.reshape(N, NP, NK)
    idx3 = src3 * NK + jnp.arange(NK, dtype=jnp.int32)[None, None, :]
    idxp = jnp.transpose(idx3, (1, 0, 2)).reshape(Q, NK)
    idxp = jnp.concatenate(
        [idxp, jnp.zeros((QPAD - Q, NK), jnp.int32)], axis=0)
    idx_flat = idxp.reshape(-1)

    def wconv(W, F):
        Wr = W.reshape(NK, F, NF).transpose(1, 0, 2)
        Wr = jnp.pad(Wr, ((0, 0), (0, 0), (0, FP - NF)))
        return Wr.reshape(F, NK * FP)

    W1r = jnp.pad(wconv(W1, N_OCC), ((0, 8 - N_OCC), (0, 0)))
    xp = jnp.pad(x, ((0, 0), (0, 8 - N_OCC)))
    W2r = jnp.pad(wconv(W2, NF), ((0, FP - NF), (0, 0)))

    b1p, g1p, be1p = _pad_row(b1, FP), _pad_row(g1, FP), _pad_row(be1, FP)
    b2p, g2p, be2p = _pad_row(b2, FP), _pad_row(g2, FP), _pad_row(be2, FP)
    Wcp = jnp.pad(Wc, ((0, FP - NF), (0, SFP - SF)))
    bcp, gcp, becp = _pad_row(bc, SFP), _pad_row(gc, SFP), _pad_row(bec, SFP)
    Wlp = jnp.pad(Wl, ((0, SFP - SF), (0, SFP - SF)))
    blp = _pad_row(bl, SFP)
    wfb = jnp.zeros((8, SFP), jnp.float32)
    wfb = wfb.at[0, :SF].set(Wf[:, 0]).at[1, 0].set(bf[0])

    G1 = pl.pallas_call(
        _mm_a_body,
        grid=(NTILES,),
        in_specs=[pl.BlockSpec((TN, 8), lambda i: (i, 0)),
                  pl.BlockSpec((8, NK * FP), lambda i: (0, 0))],
        out_specs=pl.BlockSpec((TN, NK * FP), lambda i: (i, 0)),
        out_shape=jax.ShapeDtypeStruct((N, NK * FP), jnp.bfloat16),
    )(xp, W1r)

    X1 = _gather_sum(G1.reshape(N * NK, FP), idx_flat)

    G2 = pl.pallas_call(
        _bn_mm_body,
        grid=(NTILES,),
        in_specs=_x_specs() + [_vec_spec()] * 3
        + [pl.BlockSpec((FP, NK * FP), lambda i: (0, 0))],
        out_specs=pl.BlockSpec((TN, NK * FP), lambda i: (i, 0)),
        out_shape=jax.ShapeDtypeStruct((N, NK * FP), jnp.bfloat16),
    )(X1, X1, X1, X1, X1, X1, b1p, g1p, be1p, W2r)

    X2 = _gather_sum(G2.reshape(N * NK, FP), idx_flat)

    sfv = pl.BlockSpec((8, SFP), lambda i: (0, 0))
    out = pl.pallas_call(
        _head_body,
        grid=(NTILES,),
        in_specs=_x_specs() + [_vec_spec()] * 3
        + [pl.BlockSpec((FP, SFP), lambda i: (0, 0)), sfv, sfv, sfv,
           pl.BlockSpec((SFP, SFP), lambda i: (0, 0)), sfv, sfv],
        out_specs=pl.BlockSpec((8, 128), lambda i: (0, 0)),
        out_shape=jax.ShapeDtypeStruct((8, 128), jnp.float32),
        scratch_shapes=[pltpu.VMEM((8, 128), jnp.float32)],
    )(X2, X2, X2, X2, X2, X2, b2p, g2p, be2p,
      Wcp, bcp, gcp, becp, Wlp, blp, wfb)

    return out[0:1, 0]
